# baseline (speedup 1.0000x reference)
import functools

import jax
import jax.numpy as jnp
from jax import lax
from jax.experimental import pallas as pl
from jax.experimental.pallas import tpu as pltpu
from jax.experimental.pallas import tpu_sc as plsc

NC = 2
NS = 16
NW = NC * NS
L = 16

D = 64
CB = 128
PIT = 133
C = 256


def _make_prep(vocab):
    SB = 4 * CB
    nsb = vocab // SB
    per = nsb // NW
    ntail = nsb - per * NW

    mesh = plsc.VectorSubcoreMesh(core_axis_name="c", subcore_axis_name="s")

    @functools.partial(
        pl.kernel,
        mesh=mesh,
        out_type=jax.ShapeDtypeStruct((vocab // 2, 2 * D), jnp.float32),
        scratch_types=[
            pltpu.VMEM((D, SB + 8), jnp.float32),
            pltpu.VMEM((D, SB + 8), jnp.float32),
            pltpu.VMEM((CB // 2, 2 * D), jnp.float32),
            pltpu.VMEM((CB // 2, 2 * D), jnp.float32),
            pltpu.VMEM((D, D), jnp.float32),
            pltpu.SemaphoreType.DMA,
            pltpu.SemaphoreType.DMA,
            pltpu.SemaphoreType.DMA,
            pltpu.SemaphoreType.DMA,
        ],
        compiler_params=pltpu.CompilerParams(needs_layout_passes=False),
    )
    def prep(tabT_hbm, tail_hbm, tab2_hbm, in0, in1, t0, t1, tin,
             is0, is1, os0, os1):
        wid = lax.axis_index("s") * NC + lax.axis_index("c")
        j0 = wid * per
        IN = (in0, in1)
        T = (t0, t1)
        isem = (is0, is1)
        osem = (os0, os0)

        iot = lax.iota(jnp.int32, L)

        def start_in(j, b):
            pltpu.async_copy(tabT_hbm.at[:, pl.ds(j * SB, SB)],
                             IN[b].at[:, pl.ds(0, SB)], isem[b])

        def wait_in(b):
            pltpu.make_async_copy(tabT_hbm.at[:, pl.ds(0, SB)],
                                  IN[b].at[:, pl.ds(0, SB)], isem[b]).wait()

        def transpose(tb, src, co, nrow=CB // 2):
            @plsc.parallel_loop(0, nrow, 1, unroll=8)
            def _(q):
                c0 = jnp.broadcast_to(co + 2 * q, (L,))
                c1 = jnp.broadcast_to(co + 2 * q + 1, (L,))
                for e0 in range(0, D, L):
                    T[tb][q, pl.ds(e0, L)] = plsc.load_gather(
                        src, [e0 + iot, c0])
                    T[tb][q, pl.ds(D + e0, L)] = plsc.load_gather(
                        src, [e0 + iot, c1])

        def start_out(p0, tb, nrow=CB // 2):
            pltpu.async_copy(T[tb].at[pl.ds(0, nrow)],
                             tab2_hbm.at[pl.ds(p0, nrow)], osem[tb])

        def wait_out(tb, nrow=CB // 2):
            pltpu.make_async_copy(
                T[tb].at[pl.ds(0, nrow)],
                tab2_hbm.at[pl.ds(0, nrow)], osem[tb]).wait()

        def do_sb(j, b, first):
            for u in range(4):
                tb = u % 2
                if not (first and u < 2):
                    wait_out(tb)
                transpose(tb, IN[b], u * CB)
                start_out((j * 4 + u) * (CB // 2), tb)

        start_in(j0, 0)
        start_in(j0 + 1, 1)

        wait_in(0)
        do_sb(j0, 0, True)
        start_in(j0 + 2, 0)
        wait_in(1)
        do_sb(j0 + 1, 1, False)
        start_in(j0 + 3, 1)

        def body(i2, carry):
            for b in (0, 1):
                jj = 2 * i2 + b
                wait_in(b)
                do_sb(j0 + jj, b, False)

                @pl.when(jj + 2 < per)
                def _():
                    start_in(j0 + jj + 2, b)
            return carry

        lax.fori_loop(1, per // 2, body, 0)
        wait_in(0)
        do_sb(j0 + per - 1, 0, False)
        wait_out(0)
        wait_out(1)

        @pl.when(wid == 0)
        def _():
            start_in(NW * per, 0)
            wait_in(0)
            do_sb(NW * per, 0, True)
            wait_out(0)
            wait_out(1)
            pltpu.async_copy(tail_hbm, tin, isem[0])
            pltpu.make_async_copy(tail_hbm, tin, isem[0]).wait()
            transpose(0, tin, 0, nrow=D // 2)
            start_out((vocab - D) // 2, 0, nrow=D // 2)
            wait_out(0, nrow=D // 2)

    return prep


def _make_gather(n_flat):
    bpw = n_flat // NW
    nch = bpw // C
    assert nch % 2 == 0 and nch >= 4

    mesh = plsc.VectorSubcoreMesh(core_axis_name="c", subcore_axis_name="s")

    @functools.partial(
        pl.kernel,
        mesh=mesh,
        out_type=jax.ShapeDtypeStruct((n_flat, D), jnp.float32),
        scratch_types=[
            pltpu.VMEM((C,), jnp.int32),
            pltpu.VMEM((C,), jnp.int32),
            pltpu.VMEM((C,), jnp.int32),
            pltpu.VMEM((C,), jnp.int32),
            pltpu.VMEM((C,), jnp.int32),
            pltpu.VMEM((C,), jnp.int32),
            pltpu.VMEM((C, 2 * D), jnp.float32),
            pltpu.VMEM((C, 2 * D), jnp.float32),
            pltpu.VMEM((C, D), jnp.float32),
            pltpu.SemaphoreType.DMA,
            pltpu.SemaphoreType.DMA,
            pltpu.SemaphoreType.DMA,
            pltpu.SemaphoreType.DMA,
            pltpu.SemaphoreType.DMA,
        ],
        compiler_params=pltpu.CompilerParams(needs_layout_passes=False),
    )
    def gat(tok_hbm, tab2_hbm, out_hbm, tc0, tc1, p0, p1, pv0, pv1,
            g0, g1, cb0, ts0, ts1, gs0, gs1, os0):
        wid = lax.axis_index("s") * NC + lax.axis_index("c")
        base = wid * bpw
        TC = (tc0, tc1)
        P = (p0, p1)
        PV = (pv0, pv1)
        G = (g0, g1)
        CBF = (cb0, cb0)
        tsem = (ts0, ts1)
        gsem = (gs0, gs1)
        osem = (os0, os0)

        def start_tok(i, b):
            pltpu.async_copy(tok_hbm.at[pl.ds(base + i * C, C)], TC[b],
                             tsem[b])

        def start_gather(i, b):
            pltpu.make_async_copy(tok_hbm.at[pl.ds(base, C)], TC[b],
                                  tsem[b]).wait()

            def mk(k, carry):
                tv = TC[b][pl.ds(k * L, L)]
                P[b][pl.ds(k * L, L)] = lax.shift_right_logical(tv, 1)
                PV[b][pl.ds(k * L, L)] = lax.rem(tv, 2)
                return carry
            lax.fori_loop(0, C // L, mk, 0, unroll=4)
            pltpu.async_copy(tab2_hbm.at[P[b]], G[b], gsem[b])

        def wait_gather(b):
            pltpu.make_async_copy(tab2_hbm.at[P[b]], G[b], gsem[b]).wait()

        def compact(b):
            @plsc.parallel_loop(0, C, 1, unroll=8)
            def _(t):
                g = lax.div(t, L)
                j = lax.rem(t, L)
                parv = PV[b][pl.ds(g * L, L)]
                pred = jnp.take(parv, jnp.broadcast_to(j, (L,))) == 1
                for k in range(0, D, L):
                    lo = G[b][t, pl.ds(k, L)]
                    hi = G[b][t, pl.ds(D + k, L)]
                    CBF[b][t, pl.ds(k, L)] = jnp.where(pred, hi, lo)

        def start_out(i, b):
            pltpu.async_copy(CBF[b], out_hbm.at[pl.ds(base + i * C, C)],
                             osem[b])

        def wait_out(b):
            pltpu.make_async_copy(CBF[b],
                                  out_hbm.at[pl.ds(base, C)], osem[b]).wait()

        start_tok(0, 0)
        start_tok(1, 1)
        start_gather(0, 0)
        start_tok(2, 0)
        start_gather(1, 1)
        start_tok(3, 1)

        wait_gather(0)
        compact(0)
        start_out(0, 0)
        start_gather(2, 0)
        start_tok(4, 0)

        def body(i2, carry):
            for b in (1, 0):
                i = 2 * i2 - b
                wait_gather(b)
                wait_out(b)
                compact(b)
                start_out(i, b)

                @pl.when(i + 2 < nch)
                def _():
                    start_gather(i + 2, b)

                @pl.when(i + 4 < nch)
                def _():
                    start_tok(i + 4, b)
            return carry

        lax.fori_loop(1, nch // 2, body, 0)
        wait_gather(1)
        wait_out(1)
        compact(1)
        start_out(nch - 1, 1)
        wait_out(1)

    return gat


def kernel(tokens, table):
    batch, seq = tokens.shape
    n = batch * seq
    vocab = table.shape[0]
    flat = tokens.reshape(n)
    tab2 = table.reshape(vocab // 2, 2 * D)
    out = _make_gather(n)(flat, tab2)
    return out.reshape(batch, seq, D)


# speedup vs baseline: 1.0058x; 1.0058x over previous
import functools

import jax
import jax.numpy as jnp
from jax import lax
from jax.experimental import pallas as pl
from jax.experimental.pallas import tpu as pltpu
from jax.experimental.pallas import tpu_sc as plsc

NC = 2
NS = 16
NW = NC * NS
L = 16

D = 64
C = 160


def _make_gather(n_flat):
    bpw = n_flat // NW
    nch = bpw // C
    assert nch % 2 == 0 and nch >= 4

    mesh = plsc.VectorSubcoreMesh(core_axis_name="c", subcore_axis_name="s")

    @functools.partial(
        pl.kernel,
        mesh=mesh,
        out_type=jax.ShapeDtypeStruct((n_flat, D), jnp.float32),
        scratch_types=[
            pltpu.VMEM((C,), jnp.int32),
            pltpu.VMEM((C,), jnp.int32),
            pltpu.VMEM((C,), jnp.int32),
            pltpu.VMEM((C,), jnp.int32),
            pltpu.VMEM((C,), jnp.int32),
            pltpu.VMEM((C,), jnp.int32),
            pltpu.VMEM((C, 2 * D), jnp.float32),
            pltpu.VMEM((C, 2 * D), jnp.float32),
            pltpu.VMEM((C, D), jnp.float32),
            pltpu.VMEM((C, D), jnp.float32),
            pltpu.SemaphoreType.DMA,
            pltpu.SemaphoreType.DMA,
            pltpu.SemaphoreType.DMA,
            pltpu.SemaphoreType.DMA,
            pltpu.SemaphoreType.DMA,
            pltpu.SemaphoreType.DMA,
        ],
        compiler_params=pltpu.CompilerParams(needs_layout_passes=False),
    )
    def gat(tok_hbm, tab2_hbm, out_hbm, tc0, tc1, p0, p1, pv0, pv1,
            g0, g1, cb0, cb1, ts0, ts1, gs0, gs1, os0, os1):
        wid = lax.axis_index("s") * NC + lax.axis_index("c")
        base = wid * bpw
        TC = (tc0, tc1)
        P = (p0, p1)
        PV = (pv0, pv1)
        G = (g0, g1)
        CBF = (cb0, cb1)
        tsem = (ts0, ts1)
        gsem = (gs0, gs1)
        osem = (os0, os1)

        def start_tok(i, b):
            pltpu.async_copy(tok_hbm.at[pl.ds(base + i * C, C)], TC[b],
                             tsem[b])

        def start_gather(i, b):
            pltpu.make_async_copy(tok_hbm.at[pl.ds(base, C)], TC[b],
                                  tsem[b]).wait()

            def mk(k, carry):
                tv = TC[b][pl.ds(k * L, L)]
                P[b][pl.ds(k * L, L)] = lax.shift_right_logical(tv, 1)
                PV[b][pl.ds(k * L, L)] = lax.rem(tv, 2)
                return carry
            lax.fori_loop(0, C // L, mk, 0, unroll=4)
            pltpu.async_copy(tab2_hbm.at[P[b]], G[b], gsem[b])

        def wait_gather(b):
            pltpu.make_async_copy(tab2_hbm.at[P[b]], G[b], gsem[b]).wait()

        def compact(b):
            @plsc.parallel_loop(0, C, 1, unroll=8)
            def _(t):
                g = lax.div(t, L)
                j = lax.rem(t, L)
                parv = PV[b][pl.ds(g * L, L)]
                pred = jnp.take(parv, jnp.broadcast_to(j, (L,))) == 1
                for k in range(0, D, L):
                    lo = G[b][t, pl.ds(k, L)]
                    hi = G[b][t, pl.ds(D + k, L)]
                    CBF[b][t, pl.ds(k, L)] = jnp.where(pred, hi, lo)

        def start_out(i, b):
            pltpu.async_copy(CBF[b], out_hbm.at[pl.ds(base + i * C, C)],
                             osem[b])

        def wait_out(b):
            pltpu.make_async_copy(CBF[b],
                                  out_hbm.at[pl.ds(base, C)], osem[b]).wait()

        start_tok(0, 0)
        start_tok(1, 1)
        start_gather(0, 0)
        start_tok(2, 0)
        start_gather(1, 1)
        start_tok(3, 1)
        for b in (0, 1):
            wait_gather(b)
            compact(b)
            start_out(b, b)
            start_gather(b + 2, b)
            start_tok(b + 4, b)

        def body(i2, carry):
            for b in (0, 1):
                i = 2 * i2 + b
                wait_gather(b)
                wait_out(b)
                compact(b)
                start_out(i, b)

                @pl.when(i + 2 < nch)
                def _():
                    start_gather(i + 2, b)

                @pl.when(i + 4 < nch)
                def _():
                    start_tok(i + 4, b)
            return carry

        lax.fori_loop(1, nch // 2, body, 0)
        wait_out(0)
        wait_out(1)

    return gat


def kernel(tokens, table):
    batch, seq = tokens.shape
    n = batch * seq
    vocab = table.shape[0]
    flat = tokens.reshape(n)
    tab2 = table.reshape(vocab // 2, 2 * D)
    out = _make_gather(n)(flat, tab2)
    return out.reshape(batch, seq, D)
